# initial kernel scaffold (unmeasured)
import jax
import jax.numpy as jnp
from jax import lax
from jax.experimental import pallas as pl
from jax.experimental.pallas import tpu as pltpu

M = 4096
HALF = M // 2


def kernel(dy, W):
    my_x = lax.axis_index("x")
    dy_blk = lax.dynamic_slice_in_dim(dy, my_x * HALF, HALF, axis=0)
    partial = lax.dot_general(
        dy_blk, W, (((1,), (1,)), ((), ())), preferred_element_type=jnp.float32
    )
    return _comm(partial)


def _comm(partial):
    half, n = partial.shape

    def body(p_ref, out_ref, yrecv_ref, red_ref,
             send_sem1, recv_sem1, send_sem2, recv_sem2, copy_sem):
        my_x = lax.axis_index("x")
        my_y = lax.axis_index("y")

        barrier = pltpu.get_barrier_semaphore()
        pl.semaphore_signal(barrier, inc=1, device_id=(my_x, 1 - my_y),
                            device_id_type=pl.DeviceIdType.MESH)
        pl.semaphore_signal(barrier, inc=1, device_id=(1 - my_x, my_y),
                            device_id_type=pl.DeviceIdType.MESH)
        pl.semaphore_wait(barrier, 2)

        rdma1 = pltpu.make_async_remote_copy(
            src_ref=p_ref, dst_ref=yrecv_ref,
            send_sem=send_sem1, recv_sem=recv_sem1,
            device_id=(my_x, 1 - my_y), device_id_type=pl.DeviceIdType.MESH)
        rdma1.start()
        rdma1.wait()
        red_ref[...] = p_ref[...] + yrecv_ref[...]

        row0 = my_x * half
        local = pltpu.make_async_copy(
            red_ref, out_ref.at[pl.ds(row0, half), :], copy_sem)
        local.start()

        rdma2 = pltpu.make_async_remote_copy(
            src_ref=red_ref, dst_ref=out_ref.at[pl.ds(row0, half), :],
            send_sem=send_sem2, recv_sem=recv_sem2,
            device_id=(1 - my_x, my_y), device_id_type=pl.DeviceIdType.MESH)
        rdma2.start()
        local.wait()
        rdma2.wait()

    return pl.pallas_call(
        body,
        out_shape=jax.ShapeDtypeStruct((M, n), jnp.float32),
        in_specs=[pl.BlockSpec(memory_space=pltpu.VMEM)],
        out_specs=pl.BlockSpec(memory_space=pltpu.MemorySpace.HBM),
        scratch_shapes=[
            pltpu.VMEM((half, n), jnp.float32),
            pltpu.VMEM((half, n), jnp.float32),
            pltpu.SemaphoreType.DMA,
            pltpu.SemaphoreType.DMA,
            pltpu.SemaphoreType.DMA,
            pltpu.SemaphoreType.DMA,
            pltpu.SemaphoreType.DMA,
        ],
        compiler_params=pltpu.CompilerParams(collective_id=0),
    )(partial)


# baseline (device time: 1006262 ns/iter reference)
import jax
import jax.numpy as jnp
from jax import lax
from jax.experimental import pallas as pl
from jax.experimental.pallas import tpu as pltpu

M = 4096
HALF = M // 2
C = 4
CH = HALF // C


def kernel(dy, W):
    my_x = lax.axis_index("x")
    dy_blk = lax.dynamic_slice_in_dim(dy, my_x * HALF, HALF, axis=0)
    partial = lax.dot_general(
        dy_blk, W, (((1,), (1,)), ((), ())), preferred_element_type=jnp.float32
    )
    return _comm(partial)


def _comm(partial):
    half, n = partial.shape

    def body(p_hbm, out_ref, p_vmem, yrecv, red,
             s1_send, s1_recv, s2_send, s2_recv, in_sem, out_sem):
        my_x = lax.axis_index("x")
        my_y = lax.axis_index("y")

        barrier = pltpu.get_barrier_semaphore()
        pl.semaphore_signal(barrier, inc=1, device_id=(my_x, 1 - my_y),
                            device_id_type=pl.DeviceIdType.MESH)
        pl.semaphore_signal(barrier, inc=1, device_id=(1 - my_x, my_y),
                            device_id_type=pl.DeviceIdType.MESH)
        pl.semaphore_wait(barrier, 2)

        for c in range(C):
            slot = c % 2
            row = c * CH
            cp_in = pltpu.make_async_copy(
                p_hbm.at[pl.ds(row, CH), :], p_vmem.at[slot], in_sem.at[c])
            cp_in.start()
            cp_in.wait()

            rdma1 = pltpu.make_async_remote_copy(
                src_ref=p_vmem.at[slot], dst_ref=yrecv.at[slot],
                send_sem=s1_send.at[c], recv_sem=s1_recv.at[c],
                device_id=(my_x, 1 - my_y),
                device_id_type=pl.DeviceIdType.MESH)
            rdma1.start()
            rdma1.wait()
            red[slot, :, :] = p_vmem[slot, :, :] + yrecv[slot, :, :]

            out_row = my_x * half + row
            cp_out = pltpu.make_async_copy(
                red.at[slot], out_ref.at[pl.ds(out_row, CH), :], out_sem.at[c])
            cp_out.start()
            rdma2 = pltpu.make_async_remote_copy(
                src_ref=red.at[slot], dst_ref=out_ref.at[pl.ds(out_row, CH), :],
                send_sem=s2_send.at[c], recv_sem=s2_recv.at[c],
                device_id=(1 - my_x, my_y),
                device_id_type=pl.DeviceIdType.MESH)
            rdma2.start()
            cp_out.wait()
            rdma2.wait()

    return pl.pallas_call(
        body,
        out_shape=jax.ShapeDtypeStruct((M, n), jnp.float32),
        in_specs=[pl.BlockSpec(memory_space=pltpu.MemorySpace.HBM)],
        out_specs=pl.BlockSpec(memory_space=pltpu.MemorySpace.HBM),
        scratch_shapes=[
            pltpu.VMEM((2, CH, n), jnp.float32),
            pltpu.VMEM((2, CH, n), jnp.float32),
            pltpu.VMEM((2, CH, n), jnp.float32),
            pltpu.SemaphoreType.DMA((C,)),
            pltpu.SemaphoreType.DMA((C,)),
            pltpu.SemaphoreType.DMA((C,)),
            pltpu.SemaphoreType.DMA((C,)),
            pltpu.SemaphoreType.DMA((C,)),
            pltpu.SemaphoreType.DMA((C,)),
        ],
        compiler_params=pltpu.CompilerParams(
            collective_id=0, vmem_limit_bytes=56 * 1024 * 1024),
    )(partial)


# device time: 679480 ns/iter; 1.4809x vs baseline; 1.4809x over previous
import jax
import jax.numpy as jnp
from jax import lax
from jax.experimental import pallas as pl
from jax.experimental.pallas import tpu as pltpu

M = 4096
HALF = M // 2
C = 8
CH = HALF // C
MESH = pl.DeviceIdType.MESH


def kernel(dy, W):
    my_x = lax.axis_index("x")
    dy_blk = lax.dynamic_slice_in_dim(dy, my_x * HALF, HALF, axis=0)
    partial = lax.dot_general(
        dy_blk, W, (((1,), (1,)), ((), ())), preferred_element_type=jnp.float32
    )
    return _comm(partial)


def _comm(partial):
    half, n = partial.shape

    def body(p_hbm, out_ref, p_vmem, yrecv, red,
             s1_send, s1_recv, s2_send, s2_recv, in_sem, out_sem):
        my_x = lax.axis_index("x")
        my_y = lax.axis_index("y")

        def stage_in(c):
            return pltpu.make_async_copy(
                p_hbm.at[pl.ds(c * CH, CH), :], p_vmem.at[c % 2], in_sem.at[c])

        def rdma1(c):
            return pltpu.make_async_remote_copy(
                src_ref=p_vmem.at[c % 2], dst_ref=yrecv.at[c % 3],
                send_sem=s1_send.at[c], recv_sem=s1_recv.at[c],
                device_id=(my_x, 1 - my_y), device_id_type=MESH)

        def cp_out(c):
            return pltpu.make_async_copy(
                red.at[c % 2],
                out_ref.at[pl.ds(my_x * half + c * CH, CH), :], out_sem.at[c])

        def rdma2(c):
            return pltpu.make_async_remote_copy(
                src_ref=red.at[c % 2],
                dst_ref=out_ref.at[pl.ds(my_x * half + c * CH, CH), :],
                send_sem=s2_send.at[c], recv_sem=s2_recv.at[c],
                device_id=(1 - my_x, my_y), device_id_type=MESH)

        barrier = pltpu.get_barrier_semaphore()
        pl.semaphore_signal(barrier, inc=1, device_id=(my_x, 1 - my_y),
                            device_id_type=MESH)
        pl.semaphore_signal(barrier, inc=1, device_id=(1 - my_x, my_y),
                            device_id_type=MESH)
        pl.semaphore_wait(barrier, 2)

        stage_in(0).start()
        stage_in(0).wait()
        rdma1(0).start()
        if C > 1:
            stage_in(1).start()

        for c in range(C):
            rdma1(c).wait_recv()
            if c + 1 < C:
                stage_in(c + 1).wait()
                rdma1(c + 1).start()
            if c >= 2:
                cp_out(c - 2).wait()
                rdma2(c - 2).wait_send()
            red[c % 2, :, :] = p_vmem[c % 2, :, :] + yrecv[c % 3, :, :]
            if c + 2 < C:
                rdma1(c).wait_send()
                stage_in(c + 2).start()
            cp_out(c).start()
            rdma2(c).start()

        for c in range(max(0, C - 2), C):
            cp_out(c).wait()
            rdma2(c).wait_send()
            rdma1(c).wait_send()
        for c in range(C):
            rdma2(c).wait_recv()

    return pl.pallas_call(
        body,
        out_shape=jax.ShapeDtypeStruct((M, n), jnp.float32),
        in_specs=[pl.BlockSpec(memory_space=pltpu.MemorySpace.HBM)],
        out_specs=pl.BlockSpec(memory_space=pltpu.MemorySpace.HBM),
        scratch_shapes=[
            pltpu.VMEM((2, CH, n), jnp.float32),
            pltpu.VMEM((3, CH, n), jnp.float32),
            pltpu.VMEM((2, CH, n), jnp.float32),
            pltpu.SemaphoreType.DMA((C,)),
            pltpu.SemaphoreType.DMA((C,)),
            pltpu.SemaphoreType.DMA((C,)),
            pltpu.SemaphoreType.DMA((C,)),
            pltpu.SemaphoreType.DMA((C,)),
            pltpu.SemaphoreType.DMA((C,)),
        ],
        compiler_params=pltpu.CompilerParams(
            collective_id=0, vmem_limit_bytes=56 * 1024 * 1024),
    )(partial)


# device time: 520407 ns/iter; 1.9336x vs baseline; 1.3057x over previous
import jax
import jax.numpy as jnp
from jax import lax
from jax.experimental import pallas as pl
from jax.experimental.pallas import tpu as pltpu

M = 4096
HALF = M // 2
NC = 8
R = HALF // NC
KB = 512
K = 8192
NK = K // KB
MESH = pl.DeviceIdType.MESH


def kernel(dy, W):
    n = W.shape[0]

    def body(dy_hbm, w_hbm, out_ref, w_buf, dy_buf, acc, yrecv, red,
             s1_send, s1_recv, s2_send, s2_recv, w_sem, dy_sem, out_sem):
        my_x = lax.axis_index("x")
        my_y = lax.axis_index("y")
        row0 = my_x * HALF

        def load(c, k, slot):
            w_cp = pltpu.make_async_copy(
                w_hbm.at[:, pl.ds(k * KB, KB)], w_buf.at[slot],
                w_sem.at[slot])
            dy_cp = pltpu.make_async_copy(
                dy_hbm.at[pl.ds(row0 + c * R, R), pl.ds(k * KB, KB)],
                dy_buf.at[slot], dy_sem.at[slot])
            return w_cp, dy_cp

        def rdma1(c):
            return pltpu.make_async_remote_copy(
                src_ref=acc.at[c % 2], dst_ref=yrecv.at[c % 4],
                send_sem=s1_send.at[c], recv_sem=s1_recv.at[c],
                device_id=(my_x, 1 - my_y), device_id_type=MESH)

        def cp_out(c):
            return pltpu.make_async_copy(
                red.at[c % 2],
                out_ref.at[pl.ds(row0 + c * R, R), :], out_sem.at[c])

        def rdma2(c):
            return pltpu.make_async_remote_copy(
                src_ref=red.at[c % 2],
                dst_ref=out_ref.at[pl.ds(row0 + c * R, R), :],
                send_sem=s2_send.at[c], recv_sem=s2_recv.at[c],
                device_id=(1 - my_x, my_y), device_id_type=MESH)

        def process_comm(j):
            rdma1(j).wait_recv()
            if j >= 2:
                cp_out(j - 2).wait()
                rdma2(j - 2).wait_send()
            red[j % 2, :, :] = acc[j % 2, :, :] + yrecv[j % 4, :, :]
            cp_out(j).start()
            rdma2(j).start()

        barrier = pltpu.get_barrier_semaphore()
        pl.semaphore_signal(barrier, inc=1, device_id=(my_x, 1 - my_y),
                            device_id_type=MESH)
        pl.semaphore_signal(barrier, inc=1, device_id=(1 - my_x, my_y),
                            device_id_type=MESH)
        pl.semaphore_wait(barrier, 2)

        for cp in load(0, 0, 0):
            cp.start()
        for c in range(NC):
            if c >= 2:
                rdma1(c - 2).wait_send()
            for cp in load(c, 1, 1):
                cp.start()
            for cp in load(c, 0, 0):
                cp.wait()
            acc[c % 2, :, :] = lax.dot_general(
                dy_buf[0], w_buf[0], (((1,), (1,)), ((), ())),
                preferred_element_type=jnp.float32)

            def k_body(k, _, c=c):
                slot = lax.rem(k, 2)
                nxt = lax.rem(k + 1, 2)

                @pl.when(k + 1 < NK)
                def _():
                    for cp in load(c, k + 1, nxt):
                        cp.start()

                for cp in load(c, k, slot):
                    cp.wait()
                acc[c % 2, :, :] += lax.dot_general(
                    dy_buf[slot], w_buf[slot], (((1,), (1,)), ((), ())),
                    preferred_element_type=jnp.float32)
                return 0

            lax.fori_loop(1, NK, k_body, 0)
            rdma1(c).start()
            if c + 1 < NC:
                for cp in load(c + 1, 0, 0):
                    cp.start()
            if c >= 1:
                process_comm(c - 1)
        process_comm(NC - 1)

        for c in range(NC - 2, NC):
            cp_out(c).wait()
            rdma2(c).wait_send()
            rdma1(c).wait_send()
        for c in range(NC):
            rdma2(c).wait_recv()

    return pl.pallas_call(
        body,
        out_shape=jax.ShapeDtypeStruct((M, n), jnp.float32),
        in_specs=[pl.BlockSpec(memory_space=pltpu.MemorySpace.HBM),
                  pl.BlockSpec(memory_space=pltpu.MemorySpace.HBM)],
        out_specs=pl.BlockSpec(memory_space=pltpu.MemorySpace.HBM),
        scratch_shapes=[
            pltpu.VMEM((2, n, KB), jnp.float32),
            pltpu.VMEM((2, R, KB), jnp.float32),
            pltpu.VMEM((2, R, n), jnp.float32),
            pltpu.VMEM((4, R, n), jnp.float32),
            pltpu.VMEM((2, R, n), jnp.float32),
            pltpu.SemaphoreType.DMA((NC,)),
            pltpu.SemaphoreType.DMA((NC,)),
            pltpu.SemaphoreType.DMA((NC,)),
            pltpu.SemaphoreType.DMA((NC,)),
            pltpu.SemaphoreType.DMA((2,)),
            pltpu.SemaphoreType.DMA((2,)),
            pltpu.SemaphoreType.DMA((NC,)),
        ],
        compiler_params=pltpu.CompilerParams(
            collective_id=0, vmem_limit_bytes=60 * 1024 * 1024),
    )(dy, W)
